# Initial kernel scaffold; baseline (speedup 1.0000x reference)
import jax
import jax.numpy as jnp
from jax.experimental import pallas as pl


def kernel(x, edge_index, W, b):
    raise NotImplementedError("write your pallas kernel here")


# speedup vs baseline: 13.2238x; 13.2238x over previous
import functools

import jax
import jax.numpy as jnp
from jax import lax
from jax.experimental import pallas as pl
from jax.experimental.pallas import tpu as pltpu, tpu_sc as plsc

N = 10000
E = 320000
D_IN = 128
D_OUT = 40
DP = 48
NPAD = 10240
NW = 32
CH = 128
NCH = 80
EPT = CH * NCH
EPAD = NW * EPT
RPT = NPAD // 16

_mesh = plsc.VectorSubcoreMesh(core_axis_name="c", subcore_axis_name="s")


@functools.partial(
    pl.kernel,
    mesh=_mesh,
    out_type=jax.ShapeDtypeStruct((2, NPAD), jnp.float32),
    scratch_types=[
        pltpu.VMEM_SHARED((NPAD,), jnp.float32),
        pltpu.VMEM((CH,), jnp.int32),
        pltpu.VMEM((CH,), jnp.float32),
    ],
    compiler_params=pltpu.CompilerParams(use_tc_tiling_on_sc=False),
)
def _deg_kernel(col3_hbm, zeros_hbm, ones_hbm, out_hbm, acc_sh, colc_v, ones_v):
    c = lax.axis_index("c")
    s = lax.axis_index("s")
    wid = c * 16 + s
    pltpu.sync_copy(zeros_hbm, acc_sh.at[pl.ds(s * RPT, RPT)])
    pltpu.sync_copy(ones_hbm, ones_v)
    plsc.subcore_barrier()

    def body(i, carry):
        pltpu.sync_copy(col3_hbm.at[wid, i], colc_v)
        pltpu.sync_copy(ones_v, acc_sh.at[colc_v], add=True)
        return carry

    lax.fori_loop(0, NCH, body, 0)
    plsc.subcore_barrier()
    pltpu.sync_copy(acc_sh.at[pl.ds(s * RPT, RPT)],
                    out_hbm.at[c, pl.ds(s * RPT, RPT)])


@functools.partial(
    pl.kernel,
    mesh=_mesh,
    out_type=jax.ShapeDtypeStruct((2, NPAD, DP), jnp.float32),
    scratch_types=[
        pltpu.VMEM_SHARED((NPAD, DP), jnp.float32),
        pltpu.VMEM((CH,), jnp.int32),
        pltpu.VMEM((CH,), jnp.int32),
        pltpu.VMEM((CH, DP), jnp.float32),
        pltpu.SemaphoreType.DMA,
    ],
    compiler_params=pltpu.CompilerParams(use_tc_tiling_on_sc=False),
)
def _hop_kernel(g_hbm, row3_hbm, col3_hbm, zeros_hbm, out_hbm,
                acc_sh, rowc_v, colc_v, rows_v, sem):
    c = lax.axis_index("c")
    s = lax.axis_index("s")
    wid = c * 16 + s
    pltpu.sync_copy(zeros_hbm, acc_sh.at[pl.ds(s * RPT, RPT)])
    plsc.subcore_barrier()

    def body(i, carry):
        pltpu.sync_copy(row3_hbm.at[wid, i], rowc_v)
        pltpu.sync_copy(col3_hbm.at[wid, i], colc_v)
        pltpu.async_copy(g_hbm.at[rowc_v], rows_v, sem).wait()
        pltpu.sync_copy(rows_v, acc_sh.at[colc_v], add=True)
        return carry

    lax.fori_loop(0, NCH, body, 0)
    plsc.subcore_barrier()
    pltpu.sync_copy(acc_sh.at[pl.ds(s * RPT, RPT)],
                    out_hbm.at[c, pl.ds(s * RPT, RPT)])


def _mm_body(x_ref, w_ref, dp_ref, g0_ref, dinv_ref):
    deg = dp_ref[0] + dp_ref[1] + 1.0
    dinv = lax.rsqrt(deg)
    xw = jnp.dot(x_ref[...], w_ref[...], preferred_element_type=jnp.float32)
    g0_ref[...] = xw * dinv
    dinv_ref[...] = dinv


def _mid_body(p_ref, g0_ref, dinv_ref, g1_ref):
    dinv = dinv_ref[...]
    g1_ref[...] = (p_ref[0] + p_ref[1] + g0_ref[...]) * (dinv * dinv)


def _out_body(p_ref, g1_ref, dinv_ref, b_ref, o_ref):
    res = (p_ref[0] + p_ref[1] + g1_ref[...]) * dinv_ref[...]
    o_ref[...] = res[:, :D_OUT] + b_ref[...]


def kernel(x, edge_index, W, b):
    f32 = jnp.float32
    row = edge_index[0].astype(jnp.int32)
    col = edge_index[1].astype(jnp.int32)
    pad = jnp.full((EPAD - E,), N, jnp.int32)
    row3 = jnp.concatenate([row, pad]).reshape(NW, NCH, CH)
    col3 = jnp.concatenate([col, pad]).reshape(NW, NCH, CH)
    W48 = jnp.pad(W.astype(f32), ((0, 0), (0, DP - D_OUT)))
    zeros_deg = jnp.zeros((RPT,), f32)
    ones_ch = jnp.ones((CH,), f32)
    zeros_rows = jnp.zeros((RPT, DP), f32)

    degparts = _deg_kernel(col3, zeros_deg, ones_ch)
    degparts = degparts.reshape(2, NPAD, 1)

    BLK = 640
    g0, dinv = pl.pallas_call(
        _mm_body,
        grid=(NPAD // BLK,),
        in_specs=[
            pl.BlockSpec((BLK, D_IN), lambda i: (i, 0)),
            pl.BlockSpec((D_IN, DP), lambda i: (0, 0)),
            pl.BlockSpec((2, BLK, 1), lambda i: (0, i, 0)),
        ],
        out_specs=[
            pl.BlockSpec((BLK, DP), lambda i: (i, 0)),
            pl.BlockSpec((BLK, 1), lambda i: (i, 0)),
        ],
        out_shape=[
            jax.ShapeDtypeStruct((NPAD, DP), f32),
            jax.ShapeDtypeStruct((NPAD, 1), f32),
        ],
    )(x, W48, degparts)

    a0 = _hop_kernel(g0, row3, col3, zeros_rows)

    g1 = pl.pallas_call(
        _mid_body,
        grid=(NPAD // BLK,),
        in_specs=[
            pl.BlockSpec((2, BLK, DP), lambda i: (0, i, 0)),
            pl.BlockSpec((BLK, DP), lambda i: (i, 0)),
            pl.BlockSpec((BLK, 1), lambda i: (i, 0)),
        ],
        out_specs=pl.BlockSpec((BLK, DP), lambda i: (i, 0)),
        out_shape=jax.ShapeDtypeStruct((NPAD, DP), f32),
    )(a0, g0, dinv)

    a1 = _hop_kernel(g1, row3, col3, zeros_rows)

    OBLK = 1000
    b2 = b.astype(f32).reshape(1, D_OUT)
    out = pl.pallas_call(
        _out_body,
        grid=(N // OBLK,),
        in_specs=[
            pl.BlockSpec((2, OBLK, DP), lambda i: (0, i, 0)),
            pl.BlockSpec((OBLK, DP), lambda i: (i, 0)),
            pl.BlockSpec((OBLK, 1), lambda i: (i, 0)),
            pl.BlockSpec((1, D_OUT), lambda i: (0, 0)),
        ],
        out_specs=pl.BlockSpec((OBLK, D_OUT), lambda i: (i, 0)),
        out_shape=jax.ShapeDtypeStruct((N, D_OUT), f32),
    )(a1, g1, dinv, b2)
    return out
